# Initial kernel scaffold; baseline (speedup 1.0000x reference)
import jax
import jax.numpy as jnp
from jax.experimental import pallas as pl


def kernel(xyz):
    raise NotImplementedError("write your pallas kernel here")


# speedup vs baseline: 19.8716x; 19.8716x over previous
import jax
import jax.numpy as jnp
from jax import lax
from jax.experimental import pallas as pl
from jax.experimental.pallas import tpu as pltpu
from jax.experimental.pallas import tpu_sc as plsc

N = 8192
CUT2 = 25.0
MAXP = 250000
OUTP = MAXP + 16
NW = 32
ROWS_W = N // NW
CAPW = 12288
RB = 512
CB = 2048
WPR = N // 16
WSIZES = (8192, 4096, 2048, 1024, 512, 256, 128, 64, 32, 16)


def _tc_mask_pack(a_ref, b_ref, sqr_ref, sqc_ref, p_ref, w_ref):
    r = pl.program_id(0)
    c = pl.program_id(1)

    @pl.when(r >= 4 * (c + 1))
    def _zero():
        w_ref[...] = jnp.zeros_like(w_ref)

    @pl.when(r < 4 * (c + 1))
    def _compute():
        dot = jnp.dot(a_ref[...], b_ref[...],
                      preferred_element_type=jnp.float32)
        dist2 = (sqr_ref[...] + sqc_ref[...]) - 2.0 * dot
        m = dist2 < CUT2
        ii = RB * r + lax.broadcasted_iota(jnp.int32, (RB, CB), 0)
        jj = CB * c + lax.broadcasted_iota(jnp.int32, (RB, CB), 1)
        m = jnp.logical_and(m, ii < jj)
        mb = m.astype(jnp.bfloat16)
        w = jnp.dot(mb, p_ref[...], preferred_element_type=jnp.float32)
        w_ref[...] = w.astype(jnp.int32)


def _sc_extract(words_hbm, counts_hbm, pi_hbm, pj_hbm,
                wchunk, tmpv, tmpj, pi_v, pj_v, cnt_v):
    wid = lax.axis_index("s") * 2 + lax.axis_index("c")
    iota = lax.iota(jnp.int32, 16)

    def chunk_body(ch, ptr):
        rowbase = wid * ROWS_W + ch * 32
        pltpu.sync_copy(words_hbm.at[pl.ds(rowbase, 32)], wchunk)

        def row_body(rr, ptr):
            i_s = rowbase + rr

            def grp_body(g, ptr):
                wvec = wchunk[rr, pl.ds(g * 16, 16)]
                m = wvec != 0
                scnt = plsc.all_reduce_population_count(m)[0]

                def extract(ptr):
                    jb = (g * 16 + iota) * 16
                    plsc.store_compressed(tmpv.at[pl.ds(0, 16)], wvec, mask=m)
                    plsc.store_compressed(tmpj.at[pl.ds(0, 16)], jb, mask=m)

                    def word_body(k, ptr):
                        wv = tmpv[pl.ds(k, 16)][0]
                        jb0 = tmpj[pl.ds(k, 16)][0]
                        bits = jnp.full((16,), wv, jnp.int32) >> iota
                        bm = (bits & 1) == 1
                        plsc.store_compressed(
                            pj_v.at[pl.ds(ptr, 16)], jb0 + iota, mask=bm)
                        plsc.store_compressed(
                            pi_v.at[pl.ds(ptr, 16)],
                            jnp.full((16,), i_s, jnp.int32), mask=bm)
                        nb = plsc.all_reduce_population_count(bm)[0]
                        return jnp.minimum(ptr + nb, CAPW - 16)

                    return lax.fori_loop(0, scnt, word_body, ptr)

                return lax.cond(scnt > 0, extract, lambda p: p, ptr)

            return lax.fori_loop(0, 32, grp_body, ptr)

        return lax.fori_loop(0, 32, row_body, ptr)

    total = lax.fori_loop(0, ROWS_W // 32, chunk_body, jnp.int32(0))
    cnt_v[:] = jnp.full((16,), total, jnp.int32)
    pltpu.sync_copy(cnt_v, counts_hbm.at[wid])
    pltpu.sync_copy(pi_v, pi_hbm.at[wid])
    pltpu.sync_copy(pj_v, pj_hbm.at[wid])


def _rsqrt_nr(d2):
    h = 0.5 * d2
    ibits = plsc.bitcast(d2, jnp.int32)
    y = plsc.bitcast(jnp.int32(0x5F3759DF) - (ibits >> 1), jnp.float32)
    y = y * (1.5 - h * y * y)
    y = y * (1.5 - h * y * y)
    y = y * (1.5 - h * y * y)
    y = y * (1.5 - h * y * y)
    return y


def _sc_finalize(counts_hbm, pil_hbm, pjl_hbm, x_hbm, y_hbm, z_hbm,
                 pi_hbm, pj_hbm, dl_hbm, ds_hbm, np_hbm,
                 xv, yv, zv, piv, pjv, cntv,
                 spi, spj, sds, srows,
                 w128a, w128b, hpi_b, hpj_b, hds_b, hrows, npv, sem):
    wid = lax.axis_index("s") * 2 + lax.axis_index("c")
    iota = lax.iota(jnp.int32, 16)
    zf16 = jnp.zeros((16,), jnp.float32)
    col0 = jnp.zeros((16,), jnp.int32)

    pltpu.sync_copy(x_hbm, xv)
    pltpu.sync_copy(y_hbm, yv)
    pltpu.sync_copy(z_hbm, zv)
    pltpu.sync_copy(counts_hbm, cntv)
    pltpu.sync_copy(pil_hbm.at[wid], piv.at[pl.ds(0, CAPW)])
    pltpu.sync_copy(pjl_hbm.at[wid], pjv.at[pl.ds(0, CAPW)])

    def acc(v, carry):
        off, tot = carry
        cv = cntv[v][0]
        return off + jnp.where(v < wid, cv, 0), tot + cv

    off, total = lax.fori_loop(0, NW, acc, (jnp.int32(0), jnp.int32(0)))
    mycnt = cntv[wid][0]

    @pl.when(wid == 0)
    def _npairs():
        npv[:] = jnp.full((16,), total, jnp.int32)
        pltpu.sync_copy(npv, np_hbm)

    def pair_delta(ivec, jvec, mk):
        iv = jnp.where(mk, ivec, 0)
        jv = jnp.where(mk, jvec, 0)
        xi = plsc.load_gather(xv.at[:], [iv], mask=mk)
        xj = plsc.load_gather(xv.at[:], [jv], mask=mk)
        yi = plsc.load_gather(yv.at[:], [iv], mask=mk)
        yj = plsc.load_gather(yv.at[:], [jv], mask=mk)
        zi = plsc.load_gather(zv.at[:], [iv], mask=mk)
        zj = plsc.load_gather(zv.at[:], [jv], mask=mk)
        dx = jnp.where(mk, xi - xj, 0.0)
        dy = jnp.where(mk, yi - yj, 0.0)
        dz = jnp.where(mk, zi - zj, 0.0)
        d2 = dx * dx + dy * dy + dz * dz
        d2s = jnp.maximum(d2, 1e-30)
        dist = jnp.where(mk, d2s * _rsqrt_nr(d2s), 0.0)
        return dx, dy, dz, dist

    h = (16 - (off & 15)) & 15

    def step(s, _):
        k0 = s * 16
        mk = (h + k0 + iota) < mycnt
        ivec = piv[pl.ds(h + k0, 16)]
        jvec = pjv[pl.ds(h + k0, 16)]
        dx, dy, dz, dist = pair_delta(ivec, jvec, mk)
        spi[pl.ds(k0, 16)] = ivec
        spj[pl.ds(k0, 16)] = jvec
        sds[pl.ds(k0, 16)] = dist
        lane3 = (k0 + iota) * 3
        plsc.store_scatter(srows.at[:], [lane3], dx)
        plsc.store_scatter(srows.at[:], [lane3 + 1], dy)
        plsc.store_scatter(srows.at[:], [lane3 + 2], dz)
        return 0

    lax.fori_loop(0, jnp.maximum(mycnt - h + 15, 0) // 16, step, 0)

    def windows(m, lbase, gbase):
        cur = jnp.int32(0)
        for S in WSIZES:
            bit = (m & S) != 0

            @pl.when(bit)
            def _w(cur=cur, S=S):
                ls = pl.multiple_of(lbase + cur, 16)
                gs = pl.multiple_of(gbase + cur, 16)
                pltpu.sync_copy(spi.at[pl.ds(ls, S)], pi_hbm.at[pl.ds(gs, S)])
                pltpu.sync_copy(spj.at[pl.ds(ls, S)], pj_hbm.at[pl.ds(gs, S)])
                pltpu.sync_copy(sds.at[pl.ds(ls, S)], ds_hbm.at[pl.ds(gs, S)])
                ls3 = pl.multiple_of(ls * 3, 16)
                gs3 = pl.multiple_of(gs * 3, 16)
                pltpu.sync_copy(srows.at[pl.ds(ls3, S * 3)],
                                dl_hbm.at[pl.ds(gs3, S * 3)])

            cur = cur + jnp.where(bit, S, 0)

    mlen = jnp.maximum(mycnt - ((off + mycnt) & 15) - h, 0)
    windows(mlen, 0, off + h)

    def build_head(hb0):
        p_vec = hb0 + iota

        def own(v, carry):
            owner, obase, offv = carry
            cv = cntv[v][0]
            upd = p_vec >= offv
            owner = jnp.where(upd, v, owner)
            obase = jnp.where(upd, offv, obase)
            return owner, obase, offv + cv

        owner, obase, _ = lax.fori_loop(
            0, NW, own,
            (jnp.zeros((16,), jnp.int32), jnp.zeros((16,), jnp.int32),
             jnp.int32(0)))
        loc = p_vec - obase

        def fetch(t, acc):
            hpi, hpj = acc

            def do(acc2):
                hpi, hpj = acc2
                o_t = owner[t]
                l_t = loc[t]
                fl = pl.multiple_of((l_t // 128) * 128, 128)
                pltpu.sync_copy(pil_hbm.at[o_t, pl.ds(fl, 128)],
                                w128a.at[pl.ds(0, 128)])
                pltpu.sync_copy(pjl_hbm.at[o_t, pl.ds(fl, 128)],
                                w128b.at[pl.ds(0, 128)])
                pv = w128a[pl.ds(l_t - fl, 16)][0]
                qv = w128b[pl.ds(l_t - fl, 16)][0]
                hpi = jnp.where(iota == t, pv, hpi)
                hpj = jnp.where(iota == t, qv, hpj)
                return hpi, hpj

            return lax.cond(hb0 + t < total, do, lambda a: a, (hpi, hpj))

        hpi = jnp.full((16,), -1, jnp.int32)
        hpj = jnp.full((16,), -1, jnp.int32)
        for t in range(16):
            hpi, hpj = fetch(t, (hpi, hpj))

        mk = hpi >= 0
        dx, dy, dz, dist = pair_delta(hpi, hpj, mk)
        hpi_b[:] = hpi
        hpj_b[:] = hpj
        hds_b[:] = dist
        plsc.store_scatter(hrows.at[:], [iota * 3], dx)
        plsc.store_scatter(hrows.at[:], [iota * 3 + 1], dy)
        plsc.store_scatter(hrows.at[:], [iota * 3 + 2], dz)
        hb0a = pl.multiple_of(hb0, 16)
        hb0a3 = pl.multiple_of(hb0a * 3, 16)
        pltpu.sync_copy(hpi_b, pi_hbm.at[pl.ds(hb0a, 16)])
        pltpu.sync_copy(hpj_b, pj_hbm.at[pl.ds(hb0a, 16)])
        pltpu.sync_copy(hds_b, ds_hbm.at[pl.ds(hb0a, 16)])
        pltpu.sync_copy(hrows, dl_hbm.at[pl.ds(hb0a3, 48)])

    build_head(off - (off & 15))

    @pl.when(wid == NW - 1)
    def _padhead():
        build_head(total - (total & 15))

    padbase = total + ((16 - (total & 15)) & 15)
    plen_all = OUTP - padbase
    span = ((plen_all + 511) // 512) * 16
    pb = padbase + wid * span
    plen = jnp.clip(jnp.minimum(pb + span, OUTP) - pb, 0, None)

    def fill(s, _):
        k0 = s * 16
        spi[pl.ds(k0, 16)] = jnp.full((16,), -1, jnp.int32)
        spj[pl.ds(k0, 16)] = jnp.full((16,), -1, jnp.int32)
        sds[pl.ds(k0, 16)] = zf16
        srows[pl.ds(k0 * 3, 16)] = zf16
        srows[pl.ds(k0 * 3 + 16, 16)] = zf16
        srows[pl.ds(k0 * 3 + 32, 16)] = zf16
        return 0

    @pl.when(plen > 0)
    def _dopad():
        lax.fori_loop(0, plen // 16, fill, 0)
        windows(plen, 0, pb)


@jax.jit
def kernel(xyz):
    xyz = xyz.astype(jnp.float32)
    sq = jnp.sum(xyz * xyz, axis=1)
    a8 = jnp.zeros((N, 8), jnp.float32).at[:, :3].set(xyz)
    b8 = jnp.zeros((8, N), jnp.float32).at[:3, :].set(xyz.T)
    sqr = sq[:, None]
    sqc = sq[None, :]
    jj = jnp.arange(CB)
    pmat = jnp.where(jj[:, None] // 16 == jnp.arange(CB // 16)[None, :],
                     (2.0 ** (jj % 16))[:, None], 0.0).astype(jnp.bfloat16)

    words = pl.pallas_call(
        _tc_mask_pack,
        grid=(N // RB, N // CB),
        in_specs=[
            pl.BlockSpec((RB, 8), lambda r, c: (r, 0)),
            pl.BlockSpec((8, CB), lambda r, c: (0, c)),
            pl.BlockSpec((RB, 1), lambda r, c: (r, 0)),
            pl.BlockSpec((1, CB), lambda r, c: (0, c)),
            pl.BlockSpec((CB, CB // 16), lambda r, c: (0, 0)),
        ],
        out_specs=pl.BlockSpec((RB, CB // 16), lambda r, c: (r, c)),
        out_shape=jax.ShapeDtypeStruct((N, WPR), jnp.int32),
    )(a8, b8, sqr, sqc, pmat)

    mesh = plsc.VectorSubcoreMesh(core_axis_name="c", subcore_axis_name="s")
    sc_params = pltpu.CompilerParams(needs_layout_passes=False)

    extract = pl.kernel(
        _sc_extract,
        compiler_params=sc_params,
        out_type=(
            jax.ShapeDtypeStruct((NW, 16), jnp.int32),
            jax.ShapeDtypeStruct((NW, CAPW), jnp.int32),
            jax.ShapeDtypeStruct((NW, CAPW), jnp.int32),
        ),
        mesh=mesh,
        scratch_types=[
            pltpu.VMEM((32, WPR), jnp.int32),
            pltpu.VMEM((32,), jnp.int32),
            pltpu.VMEM((32,), jnp.int32),
            pltpu.VMEM((CAPW,), jnp.int32),
            pltpu.VMEM((CAPW,), jnp.int32),
            pltpu.VMEM((16,), jnp.int32),
        ],
    )
    counts, pil, pjl = extract(words)

    x = xyz[:, 0]
    y = xyz[:, 1]
    z = xyz[:, 2]

    finalize = pl.kernel(
        _sc_finalize,
        compiler_params=sc_params,
        out_type=(
            jax.ShapeDtypeStruct((OUTP,), jnp.int32),
            jax.ShapeDtypeStruct((OUTP,), jnp.int32),
            jax.ShapeDtypeStruct((OUTP * 3,), jnp.float32),
            jax.ShapeDtypeStruct((OUTP,), jnp.float32),
            jax.ShapeDtypeStruct((16,), jnp.int32),
        ),
        mesh=mesh,
        scratch_types=[
            pltpu.VMEM((N,), jnp.float32),
            pltpu.VMEM((N,), jnp.float32),
            pltpu.VMEM((N,), jnp.float32),
            pltpu.VMEM((CAPW + 32,), jnp.int32),
            pltpu.VMEM((CAPW + 32,), jnp.int32),
            pltpu.VMEM((NW, 16), jnp.int32),
            pltpu.VMEM((CAPW,), jnp.int32),
            pltpu.VMEM((CAPW,), jnp.int32),
            pltpu.VMEM((CAPW,), jnp.float32),
            pltpu.VMEM((CAPW * 3,), jnp.float32),
            pltpu.VMEM((160,), jnp.int32),
            pltpu.VMEM((160,), jnp.int32),
            pltpu.VMEM((16,), jnp.int32),
            pltpu.VMEM((16,), jnp.int32),
            pltpu.VMEM((16,), jnp.float32),
            pltpu.VMEM((48,), jnp.float32),
            pltpu.VMEM((16,), jnp.int32),
            pltpu.SemaphoreType.DMA,
        ],
    )
    pi, pj, deltas, dists, npv = finalize(counts, pil, pjl, x, y, z)

    pairs = jnp.stack([pi[:MAXP], pj[:MAXP]], axis=0)
    return (pairs, deltas[:MAXP * 3].reshape(MAXP, 3), dists[:MAXP],
            npv[0])
